# baseline (speedup 1.0000x reference)
import functools

import jax
import jax.numpy as jnp
from jax import lax
from jax.experimental import pallas as pl
from jax.experimental.pallas import tpu as pltpu
from jax.experimental.pallas import tpu_sc as plsc

BB, TT, CC = 4, 2048, 1024
EE, KK = 8, 2
FF = 2816
FSS = 5632
NN = BB * TT
AA = NN * KK
BT = 256
NBLK = AA // BT + EE
APAD = NBLK * BT
RDIM = 128


def _silu(v):
    return v * jax.nn.sigmoid(v)


def _router_body(x_ref, wg_ref, w_ref, idx_ref):
    x = x_ref[...]
    logits = jax.lax.dot_general(
        x, wg_ref[...], (((1,), (0,)), ((), ())),
        preferred_element_type=jnp.float32)
    m = jnp.max(logits, axis=-1, keepdims=True)
    p = jnp.exp(logits - m)
    p = p / jnp.sum(p, axis=-1, keepdims=True)
    lane = jax.lax.broadcasted_iota(jnp.int32, p.shape, 1)
    p1 = jnp.max(p, axis=-1, keepdims=True)
    a1 = jnp.min(jnp.where(p == p1, lane, EE), axis=-1, keepdims=True)
    p_m = jnp.where(lane == a1, -1.0, p)
    p2 = jnp.max(p_m, axis=-1, keepdims=True)
    a2 = jnp.min(jnp.where(p_m == p2, lane, EE), axis=-1, keepdims=True)
    s = p1 + p2
    w_ref[...] = jnp.concatenate([p1 / s, p2 / s], axis=1)
    idx_ref[...] = jnp.concatenate([a1, a2], axis=1)


def _router(xf):
    rbt = 2048
    return pl.pallas_call(
        _router_body,
        grid=(NN // rbt,),
        in_specs=[
            pl.BlockSpec((rbt, CC), lambda i: (i, 0)),
            pl.BlockSpec((CC, EE), lambda i: (0, 0)),
        ],
        out_specs=[
            pl.BlockSpec((rbt, KK), lambda i: (i, 0)),
            pl.BlockSpec((rbt, KK), lambda i: (i, 0)),
        ],
        out_shape=[
            jax.ShapeDtypeStruct((NN, KK), jnp.float32),
            jax.ShapeDtypeStruct((NN, KK), jnp.int32),
        ],
    )


def _dispatch_body(e_ref, pos_ref, be_ref):
    e2d = e_ref[...]
    rows = AA // RDIM
    r_i = jax.lax.broadcasted_iota(jnp.int32, (RDIM, RDIM), 0)
    c_i = jax.lax.broadcasted_iota(jnp.int32, (RDIM, RDIM), 1)
    tri_incl = (r_i <= c_i).astype(jnp.float32)
    tri_strict = (c_i < r_i).astype(jnp.float32)

    counts = []
    for e in range(EE):
        m = (e2d == e).astype(jnp.float32)
        counts.append(jnp.sum(m).astype(jnp.int32))
    padded = [((c + BT - 1) // BT) * BT for c in counts]
    starts = []
    acc = jnp.int32(0)
    for e in range(EE):
        starts.append(acc)
        acc = acc + padded[e]

    pos = jnp.zeros((RDIM, RDIM), jnp.int32)
    for e in range(EE):
        m = (e2d == e).astype(jnp.float32)
        rowcum = jax.lax.dot_general(
            m, tri_incl, (((1,), (0,)), ((), ())),
            preferred_element_type=jnp.float32)
        rowsum = rowcum[:, RDIM - 1:RDIM]
        prefix = jax.lax.dot_general(
            tri_strict, rowsum, (((1,), (0,)), ((), ())),
            preferred_element_type=jnp.float32)
        rank = (rowcum + prefix).astype(jnp.int32) - 1
        pos = jnp.where(e2d == e, starts[e] + rank, pos)
    pos_ref[...] = pos

    bvec = jax.lax.broadcasted_iota(jnp.int32, (1, RDIM), 1) * BT
    be = jnp.zeros((1, RDIM), jnp.int32)
    for e in range(EE):
        be = be + (bvec >= (starts[e] + padded[e])).astype(jnp.int32)
    be_ref[...] = jnp.minimum(be, EE - 1)


def _dispatch(idx):
    e2d = idx.reshape(RDIM, RDIM)
    return pl.pallas_call(
        _dispatch_body,
        grid=(1,),
        in_specs=[pl.BlockSpec((RDIM, RDIM), lambda i: (0, 0))],
        out_specs=[
            pl.BlockSpec((RDIM, RDIM), lambda i: (0, 0)),
            pl.BlockSpec((1, RDIM), lambda i: (0, 0)),
        ],
        out_shape=[
            jax.ShapeDtypeStruct((RDIM, RDIM), jnp.int32),
            jax.ShapeDtypeStruct((1, RDIM), jnp.int32),
        ],
    )(e2d)


def _experts_body(be_ref, x_ref, wg_ref, wu_ref, wd_ref, o_ref):
    x = x_ref[...].astype(jnp.bfloat16)
    a = jax.lax.dot_general(x, wg_ref[0], (((1,), (0,)), ((), ())),
                            preferred_element_type=jnp.float32)
    b = jax.lax.dot_general(x, wu_ref[0], (((1,), (0,)), ((), ())),
                            preferred_element_type=jnp.float32)
    h = (_silu(a) * b).astype(jnp.bfloat16)
    o_ref[...] = jax.lax.dot_general(h, wd_ref[0], (((1,), (0,)), ((), ())),
                                     preferred_element_type=jnp.float32)


def _experts(be, xs, wg_b, wu_b, wd_b):
    grid_spec = pltpu.PrefetchScalarGridSpec(
        num_scalar_prefetch=1,
        grid=(NBLK,),
        in_specs=[
            pl.BlockSpec((BT, CC), lambda i, be: (i, 0)),
            pl.BlockSpec((1, CC, FF), lambda i, be: (be[i], 0, 0)),
            pl.BlockSpec((1, CC, FF), lambda i, be: (be[i], 0, 0)),
            pl.BlockSpec((1, FF, CC), lambda i, be: (be[i], 0, 0)),
        ],
        out_specs=pl.BlockSpec((BT, CC), lambda i, be: (i, 0)),
    )
    return pl.pallas_call(
        _experts_body,
        grid_spec=grid_spec,
        out_shape=jax.ShapeDtypeStruct((APAD, CC), jnp.float32),
    )(be, xs, wg_b, wu_b, wd_b)


FST = FSS // 2


def _shared_a_body(x_ref, wsg_ref, wsu_ref, wsd_ref, o_ref):
    x = x_ref[...]
    a = jax.lax.dot_general(x, wsg_ref[...], (((1,), (0,)), ((), ())),
                            preferred_element_type=jnp.float32)
    b = jax.lax.dot_general(x, wsu_ref[...], (((1,), (0,)), ((), ())),
                            preferred_element_type=jnp.float32)
    h = (_silu(a) * b).astype(jnp.bfloat16)
    o_ref[...] = jax.lax.dot_general(h, wsd_ref[...], (((1,), (0,)), ((), ())),
                                     preferred_element_type=jnp.float32)


def _shared_b_body(x_ref, wsg_ref, wsu_ref, wsd_ref, wsgate_ref, part_ref,
                   routed_ref, o_ref):
    x = x_ref[...]
    a = jax.lax.dot_general(x, wsg_ref[...], (((1,), (0,)), ((), ())),
                            preferred_element_type=jnp.float32)
    b = jax.lax.dot_general(x, wsu_ref[...], (((1,), (0,)), ((), ())),
                            preferred_element_type=jnp.float32)
    h = (_silu(a) * b).astype(jnp.bfloat16)
    sh = part_ref[...] + jax.lax.dot_general(
        h, wsd_ref[...], (((1,), (0,)), ((), ())),
        preferred_element_type=jnp.float32)
    gl = jax.lax.dot_general(x, wsgate_ref[...], (((1,), (0,)), ((), ())),
                             preferred_element_type=jnp.float32)
    g = jax.nn.sigmoid(gl)
    o_ref[...] = routed_ref[...] + g * sh


def _shared(xf_bf, wsg_b, wsu_b, wsd_b, wsgate_b, routed):
    sbt = 512
    part0 = pl.pallas_call(
        _shared_a_body,
        grid=(NN // sbt,),
        in_specs=[
            pl.BlockSpec((sbt, CC), lambda i: (i, 0)),
            pl.BlockSpec((CC, FST), lambda i: (0, 0)),
            pl.BlockSpec((CC, FST), lambda i: (0, 0)),
            pl.BlockSpec((FST, CC), lambda i: (0, 0)),
        ],
        out_specs=pl.BlockSpec((sbt, CC), lambda i: (i, 0)),
        out_shape=jax.ShapeDtypeStruct((NN, CC), jnp.float32),
    )(xf_bf, wsg_b[:, :FST], wsu_b[:, :FST], wsd_b[:FST, :])
    return pl.pallas_call(
        _shared_b_body,
        grid=(NN // sbt,),
        in_specs=[
            pl.BlockSpec((sbt, CC), lambda i: (i, 0)),
            pl.BlockSpec((CC, FST), lambda i: (0, 0)),
            pl.BlockSpec((CC, FST), lambda i: (0, 0)),
            pl.BlockSpec((FST, CC), lambda i: (0, 0)),
            pl.BlockSpec((CC, 1), lambda i: (0, 0)),
            pl.BlockSpec((sbt, CC), lambda i: (i, 0)),
            pl.BlockSpec((sbt, CC), lambda i: (i, 0)),
        ],
        out_specs=pl.BlockSpec((sbt, CC), lambda i: (i, 0)),
        out_shape=jax.ShapeDtypeStruct((NN, CC), jnp.float32),
    )(xf_bf, wsg_b[:, FST:], wsu_b[:, FST:], wsd_b[FST:, :], wsgate_b,
      part0, routed)


SC_NC, SC_NS = 2, 16
NW = SC_NC * SC_NS
CHUNK = 32
TCH = 16


def _sc_mesh():
    return plsc.VectorSubcoreMesh(core_axis_name="c", subcore_axis_name="s")


def _scatter_body(xf_hbm, pos_hbm, xs_hbm, tid_m, pos_m, rows_v, gsem, ssem):
    wid = lax.axis_index("s") * SC_NC + lax.axis_index("c")
    a0 = wid * (AA // NW)
    nch = AA // NW // CHUNK
    i16 = lax.broadcasted_iota(jnp.int32, (16,), 0)

    pltpu.sync_copy(pos_hbm.at[pl.ds(wid * nch, nch)], pos_m)
    for it in range(nch):
        for jj in range(CHUNK // 16):
            tid_m[it, pl.ds(jj * 16, 16)] = jnp.right_shift(
                a0 + it * CHUNK + jj * 16 + i16, 1)

    def start_gather(it, slot):
        return pltpu.async_copy(xf_hbm.at[tid_m.at[it]], rows_v.at[slot],
                                gsem)

    gathers = {}
    scatters = {}
    gathers[0] = start_gather(0, 0)
    for it in range(nch):
        s = it & 1
        nxt = (it + 1) & 1
        gathers[it].wait()
        scatters[it] = pltpu.async_copy(rows_v.at[s], xs_hbm.at[pos_m.at[it]],
                                        ssem)
        if it + 1 < nch:
            if it - 1 >= 0:
                scatters[it - 1].wait()
            gathers[it + 1] = start_gather(it + 1, nxt)
    if nch >= 2:
        scatters[nch - 2].wait()
    scatters[nch - 1].wait()


def _sc_scatter(xf, pos):
    nch = AA // NW // CHUNK
    fn = functools.partial(
        pl.kernel,
        mesh=_sc_mesh(),
        out_type=jax.ShapeDtypeStruct((APAD, CC), jnp.float32),
        scratch_types=[
            pltpu.VMEM((nch, CHUNK), jnp.int32),
            pltpu.VMEM((nch, CHUNK), jnp.int32),
            pltpu.VMEM((2, CHUNK, CC), jnp.float32),
            pltpu.SemaphoreType.DMA,
            pltpu.SemaphoreType.DMA,
        ],
    )(_scatter_body)
    return fn(xf, pos.reshape(AA // CHUNK, CHUNK))


def _combine_body(ys_hbm, pos_hbm, w_hbm, out_hbm, pos_v, w_v, rows_v,
                  out_v, gsem, osem):
    wid = lax.axis_index("s") * SC_NC + lax.axis_index("c")
    t0 = wid * (NN // NW)
    npt = NN // NW
    nch = npt // TCH

    pltpu.sync_copy(pos_hbm.at[pl.ds(2 * t0, 2 * npt)],
                    pos_v.at[pl.ds(0, 2 * npt)])
    pltpu.sync_copy(w_hbm.at[pl.ds(2 * t0, 2 * npt)],
                    w_v.at[pl.ds(0, 2 * npt)])

    def start_gather(it, slot):
        return pltpu.async_copy(
            ys_hbm.at[pos_v.at[pl.ds(2 * it * TCH, 2 * TCH)]],
            rows_v.at[slot], gsem)

    def compute(it, slot):
        def token(j, c2):
            wv = w_v[pl.ds(2 * (it * TCH + j), 16)]
            w0 = jnp.full((16,), wv[0], jnp.float32)
            w1 = jnp.full((16,), wv[1], jnp.float32)
            for cb in range(CC // 16):
                c = cb * 16
                r0 = rows_v[slot, 2 * j, pl.ds(c, 16)]
                r1 = rows_v[slot, 2 * j + 1, pl.ds(c, 16)]
                out_v[slot, j, pl.ds(c, 16)] = w0 * r0 + w1 * r1
            return c2

        lax.fori_loop(0, TCH, token, 0)

    gathers = {}
    outs = {}
    gathers[0] = start_gather(0, 0)
    for it in range(nch):
        s = it & 1
        nxt = (it + 1) & 1
        gathers[it].wait()
        if it + 1 < nch:
            if it - 1 >= 0:
                outs[it - 1].wait()
            gathers[it + 1] = start_gather(it + 1, nxt)
        compute(it, s)
        outs[it] = pltpu.async_copy(out_v.at[s],
                                    out_hbm.at[pl.ds(t0 + it * TCH, TCH)],
                                    osem)
    if nch >= 2:
        outs[nch - 2].wait()
    outs[nch - 1].wait()


def _sc_combine(ys, pos, wflat):
    fn = functools.partial(
        pl.kernel,
        mesh=_sc_mesh(),
        out_type=jax.ShapeDtypeStruct((NN, CC), jnp.float32),
        scratch_types=[
            pltpu.VMEM((2 * (NN // NW),), jnp.int32),
            pltpu.VMEM((2 * (NN // NW) + 16,), jnp.float32),
            pltpu.VMEM((2, 2 * TCH, CC), jnp.float32),
            pltpu.VMEM((2, TCH, CC), jnp.float32),
            pltpu.SemaphoreType.DMA,
            pltpu.SemaphoreType.DMA,
        ],
    )(_combine_body)
    return fn(ys, pos, wflat)


def kernel(x, Wgate, Wg, Wu, Wd, Wsg, Wsu, Wsd, Wsgate):
    xf = x.reshape(NN, CC)
    w, idx = _router(xf)(xf, Wgate)
    pos2d, be_row = _dispatch(idx)
    pos = pos2d.reshape(AA)
    be = be_row.reshape(RDIM)[:NBLK]

    xs = _sc_scatter(xf, pos)

    ys = _experts(be, xs,
                  Wg.astype(jnp.bfloat16),
                  Wu.astype(jnp.bfloat16),
                  Wd.astype(jnp.bfloat16))

    routed = _sc_combine(ys, pos, w.reshape(AA))

    out = _shared(xf.astype(jnp.bfloat16),
                  Wsg.astype(jnp.bfloat16),
                  Wsu.astype(jnp.bfloat16),
                  Wsd.astype(jnp.bfloat16),
                  Wsgate.astype(jnp.bfloat16),
                  routed)
    return out.reshape(BB, TT, CC)


# speedup vs baseline: 1.0378x; 1.0378x over previous
import functools

import jax
import jax.numpy as jnp
from jax import lax
from jax.experimental import pallas as pl
from jax.experimental.pallas import tpu as pltpu
from jax.experimental.pallas import tpu_sc as plsc

BB, TT, CC = 4, 2048, 1024
EE, KK = 8, 2
FF = 2816
FSS = 5632
NN = BB * TT
AA = NN * KK
BT = 256
NBLK = AA // BT + EE
APAD = NBLK * BT
RDIM = 128


def _silu(v):
    return v * jax.nn.sigmoid(v)


def _router_body(x_ref, wg_ref, w_ref, idx_ref):
    x = x_ref[...]
    logits = jax.lax.dot_general(
        x, wg_ref[...], (((1,), (0,)), ((), ())),
        preferred_element_type=jnp.float32)
    m = jnp.max(logits, axis=-1, keepdims=True)
    p = jnp.exp(logits - m)
    p = p / jnp.sum(p, axis=-1, keepdims=True)
    lane = jax.lax.broadcasted_iota(jnp.int32, p.shape, 1)
    p1 = jnp.max(p, axis=-1, keepdims=True)
    a1 = jnp.min(jnp.where(p == p1, lane, EE), axis=-1, keepdims=True)
    p_m = jnp.where(lane == a1, -1.0, p)
    p2 = jnp.max(p_m, axis=-1, keepdims=True)
    a2 = jnp.min(jnp.where(p_m == p2, lane, EE), axis=-1, keepdims=True)
    s = p1 + p2
    w_ref[...] = jnp.concatenate([p1 / s, p2 / s], axis=1)
    idx_ref[...] = jnp.concatenate([a1, a2], axis=1)


def _router(xf):
    rbt = 2048
    return pl.pallas_call(
        _router_body,
        grid=(NN // rbt,),
        in_specs=[
            pl.BlockSpec((rbt, CC), lambda i: (i, 0)),
            pl.BlockSpec((CC, EE), lambda i: (0, 0)),
        ],
        out_specs=[
            pl.BlockSpec((rbt, KK), lambda i: (i, 0)),
            pl.BlockSpec((rbt, KK), lambda i: (i, 0)),
        ],
        out_shape=[
            jax.ShapeDtypeStruct((NN, KK), jnp.float32),
            jax.ShapeDtypeStruct((NN, KK), jnp.int32),
        ],
    )


def _dispatch_body(e_ref, pos_ref, be_ref):
    e2d = e_ref[...]
    rows = AA // RDIM
    r_i = jax.lax.broadcasted_iota(jnp.int32, (RDIM, RDIM), 0)
    c_i = jax.lax.broadcasted_iota(jnp.int32, (RDIM, RDIM), 1)
    tri_incl = (r_i <= c_i).astype(jnp.float32)
    tri_strict = (c_i < r_i).astype(jnp.float32)

    counts = []
    for e in range(EE):
        m = (e2d == e).astype(jnp.float32)
        counts.append(jnp.sum(m).astype(jnp.int32))
    padded = [((c + BT - 1) // BT) * BT for c in counts]
    starts = []
    acc = jnp.int32(0)
    for e in range(EE):
        starts.append(acc)
        acc = acc + padded[e]

    pos = jnp.zeros((RDIM, RDIM), jnp.int32)
    for e in range(EE):
        m = (e2d == e).astype(jnp.float32)
        rowcum = jax.lax.dot_general(
            m, tri_incl, (((1,), (0,)), ((), ())),
            preferred_element_type=jnp.float32)
        rowsum = rowcum[:, RDIM - 1:RDIM]
        prefix = jax.lax.dot_general(
            tri_strict, rowsum, (((1,), (0,)), ((), ())),
            preferred_element_type=jnp.float32)
        rank = (rowcum + prefix).astype(jnp.int32) - 1
        pos = jnp.where(e2d == e, starts[e] + rank, pos)
    pos_ref[...] = pos

    bvec = jax.lax.broadcasted_iota(jnp.int32, (1, RDIM), 1) * BT
    be = jnp.zeros((1, RDIM), jnp.int32)
    for e in range(EE):
        be = be + (bvec >= (starts[e] + padded[e])).astype(jnp.int32)
    be_ref[...] = jnp.minimum(be, EE - 1)


def _dispatch(idx):
    e2d = idx.reshape(RDIM, RDIM)
    return pl.pallas_call(
        _dispatch_body,
        grid=(1,),
        in_specs=[pl.BlockSpec((RDIM, RDIM), lambda i: (0, 0))],
        out_specs=[
            pl.BlockSpec((RDIM, RDIM), lambda i: (0, 0)),
            pl.BlockSpec((1, RDIM), lambda i: (0, 0)),
        ],
        out_shape=[
            jax.ShapeDtypeStruct((RDIM, RDIM), jnp.int32),
            jax.ShapeDtypeStruct((1, RDIM), jnp.int32),
        ],
    )(e2d)


def _experts_gu_body(be_ref, x_ref, wg_ref, wu_ref, h_ref):
    x = x_ref[...].astype(jnp.bfloat16)
    a = jax.lax.dot_general(x, wg_ref[0].astype(jnp.bfloat16),
                            (((1,), (0,)), ((), ())),
                            preferred_element_type=jnp.float32)
    b = jax.lax.dot_general(x, wu_ref[0].astype(jnp.bfloat16),
                            (((1,), (0,)), ((), ())),
                            preferred_element_type=jnp.float32)
    h_ref[...] = (_silu(a) * b).astype(jnp.bfloat16)


def _experts_d_body(be_ref, h_ref, wd_ref, o_ref):
    o_ref[...] = jax.lax.dot_general(h_ref[...],
                                     wd_ref[0].astype(jnp.bfloat16),
                                     (((1,), (0,)), ((), ())),
                                     preferred_element_type=jnp.float32)


def _experts(be, xs, wg, wu, wd):
    gu_spec = pltpu.PrefetchScalarGridSpec(
        num_scalar_prefetch=1,
        grid=(NBLK,),
        in_specs=[
            pl.BlockSpec((BT, CC), lambda i, be: (i, 0)),
            pl.BlockSpec((1, CC, FF), lambda i, be: (be[i], 0, 0)),
            pl.BlockSpec((1, CC, FF), lambda i, be: (be[i], 0, 0)),
        ],
        out_specs=pl.BlockSpec((BT, FF), lambda i, be: (i, 0)),
    )
    hs = pl.pallas_call(
        _experts_gu_body,
        grid_spec=gu_spec,
        out_shape=jax.ShapeDtypeStruct((APAD, FF), jnp.bfloat16),
    )(be, xs, wg, wu)
    d_spec = pltpu.PrefetchScalarGridSpec(
        num_scalar_prefetch=1,
        grid=(NBLK,),
        in_specs=[
            pl.BlockSpec((BT, FF), lambda i, be: (i, 0)),
            pl.BlockSpec((1, FF, CC), lambda i, be: (be[i], 0, 0)),
        ],
        out_specs=pl.BlockSpec((BT, CC), lambda i, be: (i, 0)),
    )
    return pl.pallas_call(
        _experts_d_body,
        grid_spec=d_spec,
        out_shape=jax.ShapeDtypeStruct((APAD, CC), jnp.float32),
    )(be, hs, wd)


FST = FSS // 2


def _shared_a_body(x_ref, wsg_ref, wsu_ref, wsd_ref, o_ref):
    x = x_ref[...]
    a = jax.lax.dot_general(x, wsg_ref[...], (((1,), (0,)), ((), ())),
                            preferred_element_type=jnp.float32)
    b = jax.lax.dot_general(x, wsu_ref[...], (((1,), (0,)), ((), ())),
                            preferred_element_type=jnp.float32)
    h = (_silu(a) * b).astype(jnp.bfloat16)
    o_ref[...] = jax.lax.dot_general(h, wsd_ref[...], (((1,), (0,)), ((), ())),
                                     preferred_element_type=jnp.float32)


def _shared_b_body(x_ref, wsg_ref, wsu_ref, wsd_ref, wsgate_ref, part_ref,
                   routed_ref, o_ref):
    x = x_ref[...]
    a = jax.lax.dot_general(x, wsg_ref[...], (((1,), (0,)), ((), ())),
                            preferred_element_type=jnp.float32)
    b = jax.lax.dot_general(x, wsu_ref[...], (((1,), (0,)), ((), ())),
                            preferred_element_type=jnp.float32)
    h = (_silu(a) * b).astype(jnp.bfloat16)
    sh = part_ref[...] + jax.lax.dot_general(
        h, wsd_ref[...], (((1,), (0,)), ((), ())),
        preferred_element_type=jnp.float32)
    gl = jax.lax.dot_general(x, wsgate_ref[...], (((1,), (0,)), ((), ())),
                             preferred_element_type=jnp.float32)
    g = jax.nn.sigmoid(gl)
    o_ref[...] = routed_ref[...] + g * sh


def _shared(xf_bf, wsg_b, wsu_b, wsd_b, wsgate_b, routed):
    sbt = 512
    part0 = pl.pallas_call(
        _shared_a_body,
        grid=(NN // sbt,),
        in_specs=[
            pl.BlockSpec((sbt, CC), lambda i: (i, 0)),
            pl.BlockSpec((CC, FST), lambda i: (0, 0)),
            pl.BlockSpec((CC, FST), lambda i: (0, 0)),
            pl.BlockSpec((FST, CC), lambda i: (0, 0)),
        ],
        out_specs=pl.BlockSpec((sbt, CC), lambda i: (i, 0)),
        out_shape=jax.ShapeDtypeStruct((NN, CC), jnp.float32),
    )(xf_bf, wsg_b[:, :FST], wsu_b[:, :FST], wsd_b[:FST, :])
    return pl.pallas_call(
        _shared_b_body,
        grid=(NN // sbt,),
        in_specs=[
            pl.BlockSpec((sbt, CC), lambda i: (i, 0)),
            pl.BlockSpec((CC, FST), lambda i: (0, 0)),
            pl.BlockSpec((CC, FST), lambda i: (0, 0)),
            pl.BlockSpec((FST, CC), lambda i: (0, 0)),
            pl.BlockSpec((CC, 1), lambda i: (0, 0)),
            pl.BlockSpec((sbt, CC), lambda i: (i, 0)),
            pl.BlockSpec((sbt, CC), lambda i: (i, 0)),
        ],
        out_specs=pl.BlockSpec((sbt, CC), lambda i: (i, 0)),
        out_shape=jax.ShapeDtypeStruct((NN, CC), jnp.float32),
    )(xf_bf, wsg_b[:, FST:], wsu_b[:, FST:], wsd_b[FST:, :], wsgate_b,
      part0, routed)


SC_NC, SC_NS = 2, 16
NW = SC_NC * SC_NS
CHUNK = 32
TCH = 16


def _sc_mesh():
    return plsc.VectorSubcoreMesh(core_axis_name="c", subcore_axis_name="s")


def _scatter_body(xf_hbm, pos_hbm, xs_hbm, tid_m, pos_m, rows_v, gsem, ssem):
    wid = lax.axis_index("s") * SC_NC + lax.axis_index("c")
    a0 = wid * (AA // NW)
    nch = AA // NW // CHUNK
    i16 = lax.broadcasted_iota(jnp.int32, (16,), 0)

    pltpu.sync_copy(pos_hbm.at[pl.ds(wid * nch, nch)], pos_m)
    for it in range(nch):
        for jj in range(CHUNK // 16):
            tid_m[it, pl.ds(jj * 16, 16)] = jnp.right_shift(
                a0 + it * CHUNK + jj * 16 + i16, 1)

    def start_gather(it, slot):
        return pltpu.async_copy(xf_hbm.at[tid_m.at[it]], rows_v.at[slot],
                                gsem)

    gathers = {}
    scatters = {}
    gathers[0] = start_gather(0, 0)
    for it in range(nch):
        s = it & 1
        nxt = (it + 1) & 1
        gathers[it].wait()
        scatters[it] = pltpu.async_copy(rows_v.at[s], xs_hbm.at[pos_m.at[it]],
                                        ssem)
        if it + 1 < nch:
            if it - 1 >= 0:
                scatters[it - 1].wait()
            gathers[it + 1] = start_gather(it + 1, nxt)
    if nch >= 2:
        scatters[nch - 2].wait()
    scatters[nch - 1].wait()


def _sc_scatter(xf, pos):
    nch = AA // NW // CHUNK
    fn = functools.partial(
        pl.kernel,
        mesh=_sc_mesh(),
        out_type=jax.ShapeDtypeStruct((APAD, CC), jnp.float32),
        scratch_types=[
            pltpu.VMEM((nch, CHUNK), jnp.int32),
            pltpu.VMEM((nch, CHUNK), jnp.int32),
            pltpu.VMEM((2, CHUNK, CC), jnp.float32),
            pltpu.SemaphoreType.DMA,
            pltpu.SemaphoreType.DMA,
        ],
    )(_scatter_body)
    return fn(xf, pos.reshape(AA // CHUNK, CHUNK))


def _combine_body(ys_hbm, pos_hbm, w_hbm, out_hbm, pos_v, w_v, rows_v,
                  out_v, gsem, osem):
    wid = lax.axis_index("s") * SC_NC + lax.axis_index("c")
    t0 = wid * (NN // NW)
    npt = NN // NW
    nch = npt // TCH

    pltpu.sync_copy(pos_hbm.at[pl.ds(2 * t0, 2 * npt)],
                    pos_v.at[pl.ds(0, 2 * npt)])
    pltpu.sync_copy(w_hbm.at[pl.ds(2 * t0, 2 * npt)],
                    w_v.at[pl.ds(0, 2 * npt)])

    def start_gather(it, slot):
        return pltpu.async_copy(
            ys_hbm.at[pos_v.at[pl.ds(2 * it * TCH, 2 * TCH)]],
            rows_v.at[slot], gsem)

    def compute(it, slot):
        def token(j, c2):
            wv = w_v[pl.ds(2 * (it * TCH + j), 16)]
            w0 = jnp.full((16,), wv[0], jnp.float32)
            w1 = jnp.full((16,), wv[1], jnp.float32)
            for cb in range(CC // 16):
                c = cb * 16
                r0 = rows_v[slot, 2 * j, pl.ds(c, 16)]
                r1 = rows_v[slot, 2 * j + 1, pl.ds(c, 16)]
                out_v[slot, j, pl.ds(c, 16)] = w0 * r0 + w1 * r1
            return c2

        lax.fori_loop(0, TCH, token, 0)

    gathers = {}
    outs = {}
    gathers[0] = start_gather(0, 0)
    for it in range(nch):
        s = it & 1
        nxt = (it + 1) & 1
        gathers[it].wait()
        if it + 1 < nch:
            if it - 1 >= 0:
                outs[it - 1].wait()
            gathers[it + 1] = start_gather(it + 1, nxt)
        compute(it, s)
        outs[it] = pltpu.async_copy(out_v.at[s],
                                    out_hbm.at[pl.ds(t0 + it * TCH, TCH)],
                                    osem)
    if nch >= 2:
        outs[nch - 2].wait()
    outs[nch - 1].wait()


def _sc_combine(ys, pos, wflat):
    fn = functools.partial(
        pl.kernel,
        mesh=_sc_mesh(),
        out_type=jax.ShapeDtypeStruct((NN, CC), jnp.float32),
        scratch_types=[
            pltpu.VMEM((2 * (NN // NW),), jnp.int32),
            pltpu.VMEM((2 * (NN // NW) + 16,), jnp.float32),
            pltpu.VMEM((2, 2 * TCH, CC), jnp.float32),
            pltpu.VMEM((2, TCH, CC), jnp.float32),
            pltpu.SemaphoreType.DMA,
            pltpu.SemaphoreType.DMA,
        ],
    )(_combine_body)
    return fn(ys, pos, wflat)


def kernel(x, Wgate, Wg, Wu, Wd, Wsg, Wsu, Wsd, Wsgate):
    xf = x.reshape(NN, CC)
    w, idx = _router(xf)(xf, Wgate)
    pos2d, be_row = _dispatch(idx)
    pos = pos2d.reshape(AA)
    be = be_row.reshape(RDIM)[:NBLK]

    xs = _sc_scatter(xf, pos)

    ys = _experts(be, xs, Wg, Wu, Wd)

    routed = _sc_combine(ys, pos, w.reshape(AA))

    out = _shared(xf.astype(jnp.bfloat16),
                  Wsg.astype(jnp.bfloat16),
                  Wsu.astype(jnp.bfloat16),
                  Wsd.astype(jnp.bfloat16),
                  Wsgate.astype(jnp.bfloat16),
                  routed)
    return out.reshape(BB, TT, CC)


# speedup vs baseline: 1.0423x; 1.0043x over previous
import functools

import jax
import jax.numpy as jnp
from jax import lax
from jax.experimental import pallas as pl
from jax.experimental.pallas import tpu as pltpu
from jax.experimental.pallas import tpu_sc as plsc

BB, TT, CC = 4, 2048, 1024
EE, KK = 8, 2
FF = 2816
FSS = 5632
NN = BB * TT
AA = NN * KK
BT = 256
NBLK = AA // BT + EE
APAD = NBLK * BT
RDIM = 128


def _silu(v):
    return v * jax.nn.sigmoid(v)


def _router_body(x_ref, wg_ref, w_ref, idx_ref):
    x = x_ref[...]
    logits = jax.lax.dot_general(
        x, wg_ref[...], (((1,), (0,)), ((), ())),
        preferred_element_type=jnp.float32)
    m = jnp.max(logits, axis=-1, keepdims=True)
    p = jnp.exp(logits - m)
    p = p / jnp.sum(p, axis=-1, keepdims=True)
    lane = jax.lax.broadcasted_iota(jnp.int32, p.shape, 1)
    p1 = jnp.max(p, axis=-1, keepdims=True)
    a1 = jnp.min(jnp.where(p == p1, lane, EE), axis=-1, keepdims=True)
    p_m = jnp.where(lane == a1, -1.0, p)
    p2 = jnp.max(p_m, axis=-1, keepdims=True)
    a2 = jnp.min(jnp.where(p_m == p2, lane, EE), axis=-1, keepdims=True)
    s = p1 + p2
    w_ref[...] = jnp.concatenate([p1 / s, p2 / s], axis=1)
    idx_ref[...] = jnp.concatenate([a1, a2], axis=1)


def _router(xf):
    rbt = 2048
    return pl.pallas_call(
        _router_body,
        grid=(NN // rbt,),
        in_specs=[
            pl.BlockSpec((rbt, CC), lambda i: (i, 0)),
            pl.BlockSpec((CC, EE), lambda i: (0, 0)),
        ],
        out_specs=[
            pl.BlockSpec((rbt, KK), lambda i: (i, 0)),
            pl.BlockSpec((rbt, KK), lambda i: (i, 0)),
        ],
        out_shape=[
            jax.ShapeDtypeStruct((NN, KK), jnp.float32),
            jax.ShapeDtypeStruct((NN, KK), jnp.int32),
        ],
    )


def _dispatch_body(e_ref, pos_ref, be_ref):
    e2d = e_ref[...]
    rows = AA // RDIM
    r_i = jax.lax.broadcasted_iota(jnp.int32, (RDIM, RDIM), 0)
    c_i = jax.lax.broadcasted_iota(jnp.int32, (RDIM, RDIM), 1)
    tri_incl = (r_i <= c_i).astype(jnp.float32)
    tri_strict = (c_i < r_i).astype(jnp.float32)

    counts = []
    for e in range(EE):
        m = (e2d == e).astype(jnp.float32)
        counts.append(jnp.sum(m).astype(jnp.int32))
    padded = [((c + BT - 1) // BT) * BT for c in counts]
    starts = []
    acc = jnp.int32(0)
    for e in range(EE):
        starts.append(acc)
        acc = acc + padded[e]

    pos = jnp.zeros((RDIM, RDIM), jnp.int32)
    for e in range(EE):
        m = (e2d == e).astype(jnp.float32)
        rowcum = jax.lax.dot_general(
            m, tri_incl, (((1,), (0,)), ((), ())),
            preferred_element_type=jnp.float32)
        rowsum = rowcum[:, RDIM - 1:RDIM]
        prefix = jax.lax.dot_general(
            tri_strict, rowsum, (((1,), (0,)), ((), ())),
            preferred_element_type=jnp.float32)
        rank = (rowcum + prefix).astype(jnp.int32) - 1
        pos = jnp.where(e2d == e, starts[e] + rank, pos)
    pos_ref[...] = pos

    bvec = jax.lax.broadcasted_iota(jnp.int32, (1, RDIM), 1) * BT
    be = jnp.zeros((1, RDIM), jnp.int32)
    for e in range(EE):
        be = be + (bvec >= (starts[e] + padded[e])).astype(jnp.int32)
    be_ref[...] = jnp.minimum(be, EE - 1)


def _dispatch(idx):
    e2d = idx.reshape(RDIM, RDIM)
    return pl.pallas_call(
        _dispatch_body,
        grid=(1,),
        in_specs=[pl.BlockSpec((RDIM, RDIM), lambda i: (0, 0))],
        out_specs=[
            pl.BlockSpec((RDIM, RDIM), lambda i: (0, 0)),
            pl.BlockSpec((1, RDIM), lambda i: (0, 0)),
        ],
        out_shape=[
            jax.ShapeDtypeStruct((RDIM, RDIM), jnp.int32),
            jax.ShapeDtypeStruct((1, RDIM), jnp.int32),
        ],
    )(e2d)


def _experts_gu_body(be_ref, x_ref, wg_ref, wu_ref, h_ref):
    x = x_ref[...].astype(jnp.bfloat16)
    a = jax.lax.dot_general(x, wg_ref[0].astype(jnp.bfloat16),
                            (((1,), (0,)), ((), ())),
                            preferred_element_type=jnp.float32)
    b = jax.lax.dot_general(x, wu_ref[0].astype(jnp.bfloat16),
                            (((1,), (0,)), ((), ())),
                            preferred_element_type=jnp.float32)
    h_ref[...] = (_silu(a) * b).astype(jnp.bfloat16)


def _experts_d_body(be_ref, h_ref, wd_ref, o_ref):
    o_ref[...] = jax.lax.dot_general(h_ref[...],
                                     wd_ref[0].astype(jnp.bfloat16),
                                     (((1,), (0,)), ((), ())),
                                     preferred_element_type=jnp.float32)


def _experts(be, xs, wg, wu, wd):
    gu_spec = pltpu.PrefetchScalarGridSpec(
        num_scalar_prefetch=1,
        grid=(NBLK,),
        in_specs=[
            pl.BlockSpec((BT, CC), lambda i, be: (i, 0)),
            pl.BlockSpec((1, CC, FF), lambda i, be: (be[i], 0, 0)),
            pl.BlockSpec((1, CC, FF), lambda i, be: (be[i], 0, 0)),
        ],
        out_specs=pl.BlockSpec((BT, FF), lambda i, be: (i, 0)),
    )
    hs = pl.pallas_call(
        _experts_gu_body,
        grid_spec=gu_spec,
        out_shape=jax.ShapeDtypeStruct((APAD, FF), jnp.bfloat16),
    )(be, xs, wg, wu)
    d_spec = pltpu.PrefetchScalarGridSpec(
        num_scalar_prefetch=1,
        grid=(NBLK,),
        in_specs=[
            pl.BlockSpec((BT, FF), lambda i, be: (i, 0)),
            pl.BlockSpec((1, FF, CC), lambda i, be: (be[i], 0, 0)),
        ],
        out_specs=pl.BlockSpec((BT, CC), lambda i, be: (i, 0)),
    )
    return pl.pallas_call(
        _experts_d_body,
        grid_spec=d_spec,
        out_shape=jax.ShapeDtypeStruct((APAD, CC), jnp.float32),
    )(be, hs, wd)


FST = FSS // 2


def _shared_a_body(x_ref, wsg_ref, wsu_ref, wsd_ref, o_ref):
    x = x_ref[...]
    a = jax.lax.dot_general(x, wsg_ref[...], (((1,), (0,)), ((), ())),
                            preferred_element_type=jnp.float32)
    b = jax.lax.dot_general(x, wsu_ref[...], (((1,), (0,)), ((), ())),
                            preferred_element_type=jnp.float32)
    h = (_silu(a) * b).astype(jnp.bfloat16)
    o_ref[...] = jax.lax.dot_general(h, wsd_ref[...], (((1,), (0,)), ((), ())),
                                     preferred_element_type=jnp.float32)


def _shared_b_body(x_ref, wsg_ref, wsu_ref, wsd_ref, wsgate_ref, part_ref,
                   routed_ref, o_ref):
    x = x_ref[...]
    a = jax.lax.dot_general(x, wsg_ref[...], (((1,), (0,)), ((), ())),
                            preferred_element_type=jnp.float32)
    b = jax.lax.dot_general(x, wsu_ref[...], (((1,), (0,)), ((), ())),
                            preferred_element_type=jnp.float32)
    h = (_silu(a) * b).astype(jnp.bfloat16)
    sh = part_ref[...] + jax.lax.dot_general(
        h, wsd_ref[...], (((1,), (0,)), ((), ())),
        preferred_element_type=jnp.float32)
    gl = jax.lax.dot_general(x, wsgate_ref[...], (((1,), (0,)), ((), ())),
                             preferred_element_type=jnp.float32)
    g = jax.nn.sigmoid(gl)
    o_ref[...] = routed_ref[...].astype(jnp.float32) + g * sh


def _shared(xf_bf, wsg_b, wsu_b, wsd_b, wsgate_b, routed):
    sbt = 512
    part0 = pl.pallas_call(
        _shared_a_body,
        grid=(NN // sbt,),
        in_specs=[
            pl.BlockSpec((sbt, CC), lambda i: (i, 0)),
            pl.BlockSpec((CC, FST), lambda i: (0, 0)),
            pl.BlockSpec((CC, FST), lambda i: (0, 0)),
            pl.BlockSpec((FST, CC), lambda i: (0, 0)),
        ],
        out_specs=pl.BlockSpec((sbt, CC), lambda i: (i, 0)),
        out_shape=jax.ShapeDtypeStruct((NN, CC), jnp.float32),
    )(xf_bf, wsg_b[:, :FST], wsu_b[:, :FST], wsd_b[:FST, :])
    return pl.pallas_call(
        _shared_b_body,
        grid=(NN // sbt,),
        in_specs=[
            pl.BlockSpec((sbt, CC), lambda i: (i, 0)),
            pl.BlockSpec((CC, FST), lambda i: (0, 0)),
            pl.BlockSpec((CC, FST), lambda i: (0, 0)),
            pl.BlockSpec((FST, CC), lambda i: (0, 0)),
            pl.BlockSpec((CC, 1), lambda i: (0, 0)),
            pl.BlockSpec((sbt, CC), lambda i: (i, 0)),
            pl.BlockSpec((sbt, CC), lambda i: (i, 0)),
        ],
        out_specs=pl.BlockSpec((sbt, CC), lambda i: (i, 0)),
        out_shape=jax.ShapeDtypeStruct((NN, CC), jnp.float32),
    )(xf_bf, wsg_b[:, FST:], wsu_b[:, FST:], wsd_b[FST:, :], wsgate_b,
      part0, routed)


SC_NC, SC_NS = 2, 16
NW = SC_NC * SC_NS
CHUNK = 32
TCH = 16


def _sc_mesh():
    return plsc.VectorSubcoreMesh(core_axis_name="c", subcore_axis_name="s")


def _scatter_body(xf_hbm, pos_hbm, xs_hbm, tid_m, pos_m, rows_v, gsem, ssem):
    wid = lax.axis_index("s") * SC_NC + lax.axis_index("c")
    a0 = wid * (AA // NW)
    nch = AA // NW // CHUNK
    i16 = lax.broadcasted_iota(jnp.int32, (16,), 0)

    pltpu.sync_copy(pos_hbm.at[pl.ds(wid * nch, nch)], pos_m)
    for it in range(nch):
        for jj in range(CHUNK // 16):
            tid_m[it, pl.ds(jj * 16, 16)] = jnp.right_shift(
                a0 + it * CHUNK + jj * 16 + i16, 1)

    def start_gather(it, slot):
        return pltpu.async_copy(xf_hbm.at[tid_m.at[it]], rows_v.at[slot],
                                gsem)

    gathers = {}
    scatters = {}
    gathers[0] = start_gather(0, 0)
    for it in range(nch):
        s = it & 1
        nxt = (it + 1) & 1
        gathers[it].wait()
        scatters[it] = pltpu.async_copy(rows_v.at[s], xs_hbm.at[pos_m.at[it]],
                                        ssem)
        if it + 1 < nch:
            if it - 1 >= 0:
                scatters[it - 1].wait()
            gathers[it + 1] = start_gather(it + 1, nxt)
    if nch >= 2:
        scatters[nch - 2].wait()
    scatters[nch - 1].wait()


def _sc_scatter(xf, pos):
    nch = AA // NW // CHUNK
    fn = functools.partial(
        pl.kernel,
        mesh=_sc_mesh(),
        out_type=jax.ShapeDtypeStruct((APAD, CC), jnp.float32),
        scratch_types=[
            pltpu.VMEM((nch, CHUNK), jnp.int32),
            pltpu.VMEM((nch, CHUNK), jnp.int32),
            pltpu.VMEM((2, CHUNK, CC), jnp.float32),
            pltpu.SemaphoreType.DMA,
            pltpu.SemaphoreType.DMA,
        ],
    )(_scatter_body)
    return fn(xf, pos.reshape(AA // CHUNK, CHUNK))


def _combine_body(ys_hbm, pos_hbm, w_hbm, out_hbm, pos_v, w_v, rows_v,
                  out_v, gsem, osem):
    wid = lax.axis_index("s") * SC_NC + lax.axis_index("c")
    t0 = wid * (NN // NW)
    npt = NN // NW
    nch = npt // TCH

    pltpu.sync_copy(pos_hbm.at[pl.ds(2 * t0, 2 * npt)],
                    pos_v.at[pl.ds(0, 2 * npt)])
    pltpu.sync_copy(w_hbm.at[pl.ds(2 * t0, 2 * npt)],
                    w_v.at[pl.ds(0, 2 * npt)])

    def start_gather(it, slot):
        return pltpu.async_copy(
            ys_hbm.at[pos_v.at[pl.ds(2 * it * TCH, 2 * TCH)]],
            rows_v.at[slot], gsem)

    def compute(it, slot):
        def token(j, c2):
            wv = w_v[pl.ds(2 * (it * TCH + j), 16)]
            w0 = jnp.full((16,), wv[0], jnp.float32)
            w1 = jnp.full((16,), wv[1], jnp.float32)
            for cb in range(CC // 16):
                c = cb * 16
                r0 = rows_v[slot, 2 * j, pl.ds(c, 16)]
                r1 = rows_v[slot, 2 * j + 1, pl.ds(c, 16)]
                out_v[slot, j, pl.ds(c, 16)] = w0 * r0 + w1 * r1
            return c2

        lax.fori_loop(0, TCH, token, 0)

    gathers = {}
    outs = {}
    gathers[0] = start_gather(0, 0)
    for it in range(nch):
        s = it & 1
        nxt = (it + 1) & 1
        gathers[it].wait()
        if it + 1 < nch:
            if it - 1 >= 0:
                outs[it - 1].wait()
            gathers[it + 1] = start_gather(it + 1, nxt)
        compute(it, s)
        outs[it] = pltpu.async_copy(out_v.at[s],
                                    out_hbm.at[pl.ds(t0 + it * TCH, TCH)],
                                    osem)
    if nch >= 2:
        outs[nch - 2].wait()
    outs[nch - 1].wait()


def _sc_combine(ys, pos, wflat):
    fn = functools.partial(
        pl.kernel,
        mesh=_sc_mesh(),
        out_type=jax.ShapeDtypeStruct((NN, CC), jnp.float32),
        scratch_types=[
            pltpu.VMEM((2 * (NN // NW),), jnp.int32),
            pltpu.VMEM((2 * (NN // NW) + 16,), jnp.float32),
            pltpu.VMEM((2, 2 * TCH, CC), jnp.float32),
            pltpu.VMEM((2, TCH, CC), jnp.float32),
            pltpu.SemaphoreType.DMA,
            pltpu.SemaphoreType.DMA,
        ],
    )(_combine_body)
    return fn(ys, pos, wflat)


def kernel(x, Wgate, Wg, Wu, Wd, Wsg, Wsu, Wsd, Wsgate):
    xf = x.reshape(NN, CC)
    w, idx = _router(xf)(xf, Wgate)
    pos2d, be_row = _dispatch(idx)
    pos = pos2d.reshape(AA)
    be = be_row.reshape(RDIM)[:NBLK]

    xf_bf = xf.astype(jnp.bfloat16)
    xs = _sc_scatter(xf, pos)

    ys = _experts(be, xs, Wg, Wu, Wd)

    routed = _sc_combine(ys, pos, w.reshape(AA))

    out = _shared(xf_bf,
                  Wsg.astype(jnp.bfloat16),
                  Wsu.astype(jnp.bfloat16),
                  Wsd.astype(jnp.bfloat16),
                  Wsgate.astype(jnp.bfloat16),
                  routed)
    return out.reshape(BB, TT, CC)
